# baseline (speedup 1.0000x reference)
import functools

import jax
import jax.numpy as jnp
from jax import lax
from jax.experimental import pallas as pl
from jax.experimental.pallas import tpu as pltpu
from jax.experimental.pallas import tpu_sc as plsc

N = 2048
D = 768
H = 3072
O = 768
E = 8
GB = 128
NGB = N // GB
B = 256
NBLK = 24
S = NBLK * B
NW = 32
TPW = N // NW


def _gating_body(x_ref, gwt_ref, nwt_ref, eps_ref,
                 e1_ref, e2_ref, g1_ref, g2_ref, p1_ref, p2_ref,
                 cnt_ref, loss_ref, carry, tmp):
    i = pl.program_id(0)

    @pl.when(i == 0)
    def _():
        carry[...] = jnp.zeros_like(carry)
        tmp[...] = jnp.zeros_like(tmp)

    xb = x_ref[...].astype(jnp.bfloat16)
    g = jnp.dot(xb, gwt_ref[...], preferred_element_type=jnp.float32)
    nz = jnp.dot(xb, nwt_ref[...], preferred_element_type=jnp.float32)
    sp = jnp.maximum(nz, 0.0) + jnp.log(1.0 + jnp.exp(-jnp.abs(nz)))
    h = g + eps_ref[...] * sp

    neg_inf = jnp.float32(-jnp.inf)
    m1 = jnp.min(h, axis=1, keepdims=True)
    m2 = jnp.min(jnp.where(h == m1, jnp.float32(jnp.inf), h), axis=1,
                 keepdims=True)
    h = jnp.where((h == m1) | (h == m2), neg_inf, h)

    mx = jnp.max(h, axis=1, keepdims=True)
    p = jnp.exp(h - mx)
    L = p / jnp.sum(p, axis=1, keepdims=True)
    tmp[...] += jnp.sum(L, axis=0, keepdims=True)

    lane = lax.broadcasted_iota(jnp.int32, (GB, E), 1)
    M1 = jnp.max(L, axis=1, keepdims=True)
    i1 = jnp.min(jnp.where(L == M1, lane, E), axis=1, keepdims=True)
    L2 = jnp.where(lane == i1, -1.0, L)
    M2 = jnp.max(L2, axis=1, keepdims=True)
    i2 = jnp.min(jnp.where(L2 == M2, lane, E), axis=1, keepdims=True)

    r = (jnp.equal(lane, i1) | jnp.equal(lane, i2)).astype(jnp.bfloat16)
    row = lax.broadcasted_iota(jnp.int32, (GB, GB), 0)
    col = lax.broadcasted_iota(jnp.int32, (GB, GB), 1)
    tri = (row > col).astype(jnp.bfloat16)
    posx = jnp.dot(tri, r, preferred_element_type=jnp.float32)
    pos = posx + carry[...]
    p1 = jnp.sum(jnp.where(lane == i1, pos, 0.0), axis=1, keepdims=True)
    p2 = jnp.sum(jnp.where(lane == i2, pos, 0.0), axis=1, keepdims=True)
    carry[...] += jnp.sum(r.astype(jnp.float32), axis=0, keepdims=True)

    e1_ref[...] = i1
    e2_ref[...] = i2
    g1_ref[...] = M1
    g2_ref[...] = M2
    p1_ref[...] = p1.astype(jnp.int32)
    p2_ref[...] = p2.astype(jnp.int32)
    cnt_ref[...] = carry[...].astype(jnp.int32)

    t = tmp[...]
    mu = jnp.sum(t) / E
    var = jnp.sum((t - mu) ** 2) / (E - 1)
    loss_ref[...] = jnp.full((1, 1), var / (mu * mu), jnp.float32)


def _gating(x, gwt, nwt, eps):
    tok_i = pl.BlockSpec((GB, 1), lambda i: (i, 0))
    return pl.pallas_call(
        _gating_body,
        grid=(NGB,),
        in_specs=[
            pl.BlockSpec((GB, D), lambda i: (i, 0)),
            pl.BlockSpec((D, E), lambda i: (0, 0)),
            pl.BlockSpec((D, 1), lambda i: (0, 0)),
            pl.BlockSpec((GB, E), lambda i: (i, 0)),
        ],
        out_specs=[tok_i, tok_i, tok_i, tok_i, tok_i, tok_i,
                   pl.BlockSpec((1, E), lambda i: (0, 0)),
                   pl.BlockSpec((1, 1), lambda i: (0, 0))],
        out_shape=[
            jax.ShapeDtypeStruct((N, 1), jnp.int32),
            jax.ShapeDtypeStruct((N, 1), jnp.int32),
            jax.ShapeDtypeStruct((N, 1), jnp.float32),
            jax.ShapeDtypeStruct((N, 1), jnp.float32),
            jax.ShapeDtypeStruct((N, 1), jnp.int32),
            jax.ShapeDtypeStruct((N, 1), jnp.int32),
            jax.ShapeDtypeStruct((1, E), jnp.int32),
            jax.ShapeDtypeStruct((1, 1), jnp.float32),
        ],
        scratch_shapes=[pltpu.VMEM((1, E), jnp.float32),
                        pltpu.VMEM((1, E), jnp.float32)],
    )(x, gwt, nwt, eps)


def _dispatch_sc(x, s1, s2, g1w, g2w):
    mesh = plsc.VectorSubcoreMesh(core_axis_name="c", subcore_axis_name="s")

    @functools.partial(
        pl.kernel,
        out_type=(jax.ShapeDtypeStruct((S, D), jnp.float32),
                  jax.ShapeDtypeStruct((S, 128), jnp.float32)),
        mesh=mesh,
        scratch_types=[
            pltpu.VMEM((TPW,), jnp.int32),
            pltpu.VMEM((TPW,), jnp.int32),
            pltpu.VMEM((TPW, D), jnp.float32),
            pltpu.VMEM((TPW, 128), jnp.float32),
        ],
    )
    def k(x_hbm, s1_hbm, s2_hbm, g1w_hbm, g2w_hbm, xs_hbm, wg_hbm,
          i1v, i2v, xv, gv):
        wid = lax.axis_index("s") * 2 + lax.axis_index("c")
        base = wid * TPW
        pltpu.sync_copy(s1_hbm.at[pl.ds(base, TPW)], i1v)
        pltpu.sync_copy(s2_hbm.at[pl.ds(base, TPW)], i2v)
        pltpu.sync_copy(x_hbm.at[pl.ds(base, TPW)], xv)
        pltpu.sync_copy(xv, xs_hbm.at[i1v])
        pltpu.sync_copy(xv, xs_hbm.at[i2v])
        pltpu.sync_copy(g1w_hbm.at[pl.ds(base, TPW)], gv)
        pltpu.sync_copy(gv, wg_hbm.at[i1v])
        pltpu.sync_copy(g2w_hbm.at[pl.ds(base, TPW)], gv)
        pltpu.sync_copy(gv, wg_hbm.at[i2v])

    return k(x, s1, s2, g1w, g2w)


def _ffn_body(be_ref, xs_ref, w1t_ref, b1_ref, w2t_ref, b2_ref, wg_ref, ys_ref):
    xb = xs_ref[...].astype(jnp.bfloat16)
    h = jnp.dot(xb, w1t_ref[0], preferred_element_type=jnp.float32) + b1_ref[0]
    h = jnp.maximum(h, 0.0).astype(jnp.bfloat16)
    y = jnp.dot(h, w2t_ref[0], preferred_element_type=jnp.float32) + b2_ref[0]
    ys_ref[...] = y * wg_ref[:, :1]


def _ffn(be, xs, w1t, b1, w2t, b2, wg):
    grid_spec = pltpu.PrefetchScalarGridSpec(
        num_scalar_prefetch=1,
        grid=(NBLK,),
        in_specs=[
            pl.BlockSpec((B, D), lambda b, be: (b, 0)),
            pl.BlockSpec((1, D, H), lambda b, be: (be[b], 0, 0)),
            pl.BlockSpec((1, 1, H), lambda b, be: (be[b], 0, 0)),
            pl.BlockSpec((1, H, O), lambda b, be: (be[b], 0, 0)),
            pl.BlockSpec((1, 1, O), lambda b, be: (be[b], 0, 0)),
            pl.BlockSpec((B, 128), lambda b, be: (b, 0)),
        ],
        out_specs=pl.BlockSpec((B, O), lambda b, be: (b, 0)),
    )
    return pl.pallas_call(
        _ffn_body,
        grid_spec=grid_spec,
        out_shape=jax.ShapeDtypeStruct((S, O), jnp.float32),
    )(be, xs, w1t, b1, w2t, b2, wg)


def _combine_sc(ys, s1, s2):
    mesh = plsc.VectorSubcoreMesh(core_axis_name="c", subcore_axis_name="s")

    @functools.partial(
        pl.kernel,
        out_type=jax.ShapeDtypeStruct((N, O), jnp.float32),
        mesh=mesh,
        scratch_types=[
            pltpu.VMEM((TPW,), jnp.int32),
            pltpu.VMEM((TPW,), jnp.int32),
            pltpu.VMEM((TPW, O), jnp.float32),
            pltpu.VMEM((TPW, O), jnp.float32),
        ],
    )
    def k(ys_hbm, s1_hbm, s2_hbm, out_hbm, i1v, i2v, av, bv):
        wid = lax.axis_index("s") * 2 + lax.axis_index("c")
        base = wid * TPW
        pltpu.sync_copy(s1_hbm.at[pl.ds(base, TPW)], i1v)
        pltpu.sync_copy(s2_hbm.at[pl.ds(base, TPW)], i2v)
        pltpu.sync_copy(ys_hbm.at[i1v], av)
        pltpu.sync_copy(ys_hbm.at[i2v], bv)

        @pl.loop(0, TPW)
        def _(j):
            @pl.loop(0, O, step=16)
            def _(c):
                av.at[j, pl.ds(c, 16)][...] = (av[j, pl.ds(c, 16)]
                                               + bv[j, pl.ds(c, 16)])

        pltpu.sync_copy(av, out_hbm.at[pl.ds(base, TPW)])

    return k(ys, s1, s2)


def kernel(x, gate_w, noise_w, w1, b1, w2, b2, noise_eps):
    gwt = gate_w.T.astype(jnp.bfloat16)
    nwt = noise_w.T.astype(jnp.bfloat16)
    w1t = jnp.swapaxes(w1, 1, 2).astype(jnp.bfloat16)
    w2t = jnp.swapaxes(w2, 1, 2).astype(jnp.bfloat16)

    e1, e2, g1, g2, p1, p2, counts, loss = _gating(x, gwt, nwt, noise_eps)

    cnt = counts[0]
    nblk = (cnt + (B - 1)) // B
    starts = jnp.concatenate([jnp.zeros((1,), jnp.int32),
                              jnp.cumsum(nblk)[:-1].astype(jnp.int32)])
    off = starts * B
    bids = jnp.arange(NBLK, dtype=jnp.int32)
    be = jnp.clip(jnp.sum((bids[:, None] >= starts[None, :]).astype(jnp.int32),
                          axis=1) - 1, 0, E - 1)

    s1 = (jnp.take(off, e1[:, 0]) + p1[:, 0]).astype(jnp.int32)
    s2 = (jnp.take(off, e2[:, 0]) + p2[:, 0]).astype(jnp.int32)
    g1w = jnp.broadcast_to(g1, (N, 128))
    g2w = jnp.broadcast_to(g2, (N, 128))

    xs, wg = _dispatch_sc(x, s1, s2, g1w, g2w)
    ys = _ffn(be, xs, w1t, b1.reshape(E, 1, H), w2t, b2.reshape(E, 1, O), wg)
    out = _combine_sc(ys, s1, s2)
    return out, loss[0, 0]


# speedup vs baseline: 1.3710x; 1.3710x over previous
import functools

import jax
import jax.numpy as jnp
from jax import lax
from jax.experimental import pallas as pl
from jax.experimental.pallas import tpu as pltpu
from jax.experimental.pallas import tpu_sc as plsc

N = 2048
D = 768
H = 3072
O = 768
E = 8
GB = 128
NGB = N // GB
B = 256
NBLK = 24
S = NBLK * B
NW = 32
TPW = N // NW


def _gating_body(x_ref, gwt_ref, nwt_ref, eps_ref,
                 e1_ref, e2_ref, g1_ref, g2_ref, p1_ref, p2_ref,
                 cnt_ref, loss_ref, carry, tmp):
    i = pl.program_id(0)

    @pl.when(i == 0)
    def _():
        carry[...] = jnp.zeros_like(carry)
        tmp[...] = jnp.zeros_like(tmp)

    xb = x_ref[...].astype(jnp.bfloat16)
    g = jnp.dot(xb, gwt_ref[...], preferred_element_type=jnp.float32)
    nz = jnp.dot(xb, nwt_ref[...], preferred_element_type=jnp.float32)
    sp = jnp.maximum(nz, 0.0) + jnp.log(1.0 + jnp.exp(-jnp.abs(nz)))
    h = g + eps_ref[...] * sp

    neg_inf = jnp.float32(-jnp.inf)
    m1 = jnp.min(h, axis=1, keepdims=True)
    m2 = jnp.min(jnp.where(h == m1, jnp.float32(jnp.inf), h), axis=1,
                 keepdims=True)
    h = jnp.where((h == m1) | (h == m2), neg_inf, h)

    mx = jnp.max(h, axis=1, keepdims=True)
    p = jnp.exp(h - mx)
    L = p / jnp.sum(p, axis=1, keepdims=True)
    tmp[...] += jnp.sum(L, axis=0, keepdims=True)

    lane = lax.broadcasted_iota(jnp.int32, (GB, E), 1)
    M1 = jnp.max(L, axis=1, keepdims=True)
    i1 = jnp.min(jnp.where(L == M1, lane, E), axis=1, keepdims=True)
    L2 = jnp.where(lane == i1, -1.0, L)
    M2 = jnp.max(L2, axis=1, keepdims=True)
    i2 = jnp.min(jnp.where(L2 == M2, lane, E), axis=1, keepdims=True)

    r = (jnp.equal(lane, i1) | jnp.equal(lane, i2)).astype(jnp.bfloat16)
    row = lax.broadcasted_iota(jnp.int32, (GB, GB), 0)
    col = lax.broadcasted_iota(jnp.int32, (GB, GB), 1)
    tri = (row > col).astype(jnp.bfloat16)
    posx = jnp.dot(tri, r, preferred_element_type=jnp.float32)
    pos = posx + carry[...]
    p1 = jnp.sum(jnp.where(lane == i1, pos, 0.0), axis=1, keepdims=True)
    p2 = jnp.sum(jnp.where(lane == i2, pos, 0.0), axis=1, keepdims=True)
    carry[...] += jnp.sum(r.astype(jnp.float32), axis=0, keepdims=True)

    e1_ref[...] = i1
    e2_ref[...] = i2
    g1_ref[...] = M1
    g2_ref[...] = M2
    p1_ref[...] = p1.astype(jnp.int32)
    p2_ref[...] = p2.astype(jnp.int32)
    cnt_ref[...] = carry[...].astype(jnp.int32)

    t = tmp[...]
    mu = jnp.sum(t) / E
    var = jnp.sum((t - mu) ** 2) / (E - 1)
    loss_ref[...] = jnp.full((1, 1), var / (mu * mu), jnp.float32)


def _gating(x, gwt, nwt, eps):
    tok_i = pl.BlockSpec((GB, 1), lambda i: (i, 0))
    return pl.pallas_call(
        _gating_body,
        grid=(NGB,),
        in_specs=[
            pl.BlockSpec((GB, D), lambda i: (i, 0)),
            pl.BlockSpec((D, E), lambda i: (0, 0)),
            pl.BlockSpec((D, 1), lambda i: (0, 0)),
            pl.BlockSpec((GB, E), lambda i: (i, 0)),
        ],
        out_specs=[tok_i, tok_i, tok_i, tok_i, tok_i, tok_i,
                   pl.BlockSpec((1, E), lambda i: (0, 0)),
                   pl.BlockSpec((1, 1), lambda i: (0, 0))],
        out_shape=[
            jax.ShapeDtypeStruct((N, 1), jnp.int32),
            jax.ShapeDtypeStruct((N, 1), jnp.int32),
            jax.ShapeDtypeStruct((N, 1), jnp.float32),
            jax.ShapeDtypeStruct((N, 1), jnp.float32),
            jax.ShapeDtypeStruct((N, 1), jnp.int32),
            jax.ShapeDtypeStruct((N, 1), jnp.int32),
            jax.ShapeDtypeStruct((1, E), jnp.int32),
            jax.ShapeDtypeStruct((1, 1), jnp.float32),
        ],
        scratch_shapes=[pltpu.VMEM((1, E), jnp.float32),
                        pltpu.VMEM((1, E), jnp.float32)],
    )(x, gwt, nwt, eps)


def _dispatch_sc(x, s1, s2, g1w, g2w):
    mesh = plsc.VectorSubcoreMesh(core_axis_name="c", subcore_axis_name="s")

    @functools.partial(
        pl.kernel,
        out_type=(jax.ShapeDtypeStruct((S, D), jnp.float32),
                  jax.ShapeDtypeStruct((S, 128), jnp.float32)),
        mesh=mesh,
        scratch_types=[
            pltpu.VMEM((TPW,), jnp.int32),
            pltpu.VMEM((TPW,), jnp.int32),
            pltpu.VMEM((TPW, D), jnp.float32),
            pltpu.VMEM((TPW, 128), jnp.float32),
        ],
    )
    def k(x_hbm, s1_hbm, s2_hbm, g1w_hbm, g2w_hbm, xs_hbm, wg_hbm,
          i1v, i2v, xv, gv):
        wid = lax.axis_index("s") * 2 + lax.axis_index("c")
        base = wid * TPW
        pltpu.sync_copy(s1_hbm.at[pl.ds(base, TPW)], i1v)
        pltpu.sync_copy(s2_hbm.at[pl.ds(base, TPW)], i2v)
        pltpu.sync_copy(x_hbm.at[pl.ds(base, TPW)], xv)
        pltpu.sync_copy(xv, xs_hbm.at[i1v])
        pltpu.sync_copy(xv, xs_hbm.at[i2v])
        pltpu.sync_copy(g1w_hbm.at[pl.ds(base, TPW)], gv)
        pltpu.sync_copy(gv, wg_hbm.at[i1v])
        pltpu.sync_copy(g2w_hbm.at[pl.ds(base, TPW)], gv)
        pltpu.sync_copy(gv, wg_hbm.at[i2v])

    return k(x, s1, s2, g1w, g2w)


def _ffn_body(be_ref, xs_ref, w1_ref, b1_ref, w2_ref, b2_ref, wg_ref, ys_ref):
    kt = (((1,), (1,)), ((), ()))
    xb = xs_ref[...].astype(jnp.bfloat16)
    w1b = w1_ref[0].astype(jnp.bfloat16)
    h = lax.dot_general(xb, w1b, kt, preferred_element_type=jnp.float32)
    h = jnp.maximum(h + b1_ref[0], 0.0).astype(jnp.bfloat16)
    w2b = w2_ref[0].astype(jnp.bfloat16)
    y = lax.dot_general(h, w2b, kt, preferred_element_type=jnp.float32)
    ys_ref[...] = (y + b2_ref[0]) * wg_ref[:, :1]


def _ffn(be, xs, w1t, b1, w2t, b2, wg):
    grid_spec = pltpu.PrefetchScalarGridSpec(
        num_scalar_prefetch=1,
        grid=(NBLK,),
        in_specs=[
            pl.BlockSpec((B, D), lambda b, be: (b, 0)),
            pl.BlockSpec((1, H, D), lambda b, be: (be[b], 0, 0)),
            pl.BlockSpec((1, 1, H), lambda b, be: (be[b], 0, 0)),
            pl.BlockSpec((1, O, H), lambda b, be: (be[b], 0, 0)),
            pl.BlockSpec((1, 1, O), lambda b, be: (be[b], 0, 0)),
            pl.BlockSpec((B, 128), lambda b, be: (b, 0)),
        ],
        out_specs=pl.BlockSpec((B, O), lambda b, be: (b, 0)),
    )
    return pl.pallas_call(
        _ffn_body,
        grid_spec=grid_spec,
        out_shape=jax.ShapeDtypeStruct((S, O), jnp.float32),
    )(be, xs, w1t, b1, w2t, b2, wg)


def _combine_sc(ys, s1, s2):
    mesh = plsc.VectorSubcoreMesh(core_axis_name="c", subcore_axis_name="s")

    @functools.partial(
        pl.kernel,
        out_type=jax.ShapeDtypeStruct((N, O), jnp.float32),
        mesh=mesh,
        scratch_types=[
            pltpu.VMEM((TPW,), jnp.int32),
            pltpu.VMEM((TPW,), jnp.int32),
            pltpu.VMEM((TPW, O), jnp.float32),
            pltpu.VMEM((TPW, O), jnp.float32),
        ],
    )
    def k(ys_hbm, s1_hbm, s2_hbm, out_hbm, i1v, i2v, av, bv):
        wid = lax.axis_index("s") * 2 + lax.axis_index("c")
        base = wid * TPW
        pltpu.sync_copy(s1_hbm.at[pl.ds(base, TPW)], i1v)
        pltpu.sync_copy(s2_hbm.at[pl.ds(base, TPW)], i2v)
        pltpu.sync_copy(ys_hbm.at[i1v], av)
        pltpu.sync_copy(ys_hbm.at[i2v], bv)

        @pl.loop(0, TPW)
        def _(j):
            @pl.loop(0, O, step=16)
            def _(c):
                av.at[j, pl.ds(c, 16)][...] = (av[j, pl.ds(c, 16)]
                                               + bv[j, pl.ds(c, 16)])

        pltpu.sync_copy(av, out_hbm.at[pl.ds(base, TPW)])

    return k(ys, s1, s2)


def kernel(x, gate_w, noise_w, w1, b1, w2, b2, noise_eps):
    gwt = gate_w.T.astype(jnp.bfloat16)
    nwt = noise_w.T.astype(jnp.bfloat16)

    e1, e2, g1, g2, p1, p2, counts, loss = _gating(x, gwt, nwt, noise_eps)

    cnt = counts[0]
    nblk = (cnt + (B - 1)) // B
    starts = jnp.concatenate([jnp.zeros((1,), jnp.int32),
                              jnp.cumsum(nblk)[:-1].astype(jnp.int32)])
    off = starts * B
    bids = jnp.arange(NBLK, dtype=jnp.int32)
    be = jnp.clip(jnp.sum((bids[:, None] >= starts[None, :]).astype(jnp.int32),
                          axis=1) - 1, 0, E - 1)

    s1 = (jnp.take(off, e1[:, 0]) + p1[:, 0]).astype(jnp.int32)
    s2 = (jnp.take(off, e2[:, 0]) + p2[:, 0]).astype(jnp.int32)
    g1w = jnp.broadcast_to(g1, (N, 128))
    g2w = jnp.broadcast_to(g2, (N, 128))

    xs, wg = _dispatch_sc(x, s1, s2, g1w, g2w)
    ys = _ffn(be, xs, w1, b1.reshape(E, 1, H), w2, b2.reshape(E, 1, O), wg)
    out = _combine_sc(ys, s1, s2)
    return out, loss[0, 0]


# speedup vs baseline: 1.5345x; 1.1193x over previous
import functools

import jax
import jax.numpy as jnp
from jax import lax
from jax.experimental import pallas as pl
from jax.experimental.pallas import tpu as pltpu
from jax.experimental.pallas import tpu_sc as plsc

N = 2048
D = 768
H = 3072
O = 768
E = 8
GB = 256
NGB = N // GB
B = 256
NBLK = 24
S = NBLK * B
NW = 32
TPW = N // NW


def _gating_body(x_ref, gwt_ref, nwt_ref, eps_ref,
                 e1_ref, e2_ref, g1_ref, g2_ref, p1_ref, p2_ref,
                 cnt_ref, loss_ref, carry, tmp):
    i = pl.program_id(0)

    @pl.when(i == 0)
    def _():
        carry[...] = jnp.zeros_like(carry)
        tmp[...] = jnp.zeros_like(tmp)

    xb = x_ref[...].astype(jnp.bfloat16)
    g = jnp.dot(xb, gwt_ref[...], preferred_element_type=jnp.float32)
    nz = jnp.dot(xb, nwt_ref[...], preferred_element_type=jnp.float32)
    sp = jnp.maximum(nz, 0.0) + jnp.log(1.0 + jnp.exp(-jnp.abs(nz)))
    h = g + eps_ref[...] * sp

    neg_inf = jnp.float32(-jnp.inf)
    m1 = jnp.min(h, axis=1, keepdims=True)
    m2 = jnp.min(jnp.where(h == m1, jnp.float32(jnp.inf), h), axis=1,
                 keepdims=True)
    h = jnp.where((h == m1) | (h == m2), neg_inf, h)

    mx = jnp.max(h, axis=1, keepdims=True)
    p = jnp.exp(h - mx)
    L = p / jnp.sum(p, axis=1, keepdims=True)
    tmp[...] += jnp.sum(L, axis=0, keepdims=True)

    lane = lax.broadcasted_iota(jnp.int32, (GB, E), 1)
    M1 = jnp.max(L, axis=1, keepdims=True)
    i1 = jnp.min(jnp.where(L == M1, lane, E), axis=1, keepdims=True)
    L2 = jnp.where(lane == i1, -1.0, L)
    M2 = jnp.max(L2, axis=1, keepdims=True)
    i2 = jnp.min(jnp.where(L2 == M2, lane, E), axis=1, keepdims=True)

    r = (jnp.equal(lane, i1) | jnp.equal(lane, i2)).astype(jnp.bfloat16)
    row = lax.broadcasted_iota(jnp.int32, (GB, GB), 0)
    col = lax.broadcasted_iota(jnp.int32, (GB, GB), 1)
    tri = (row > col).astype(jnp.bfloat16)
    posx = jnp.dot(tri, r, preferred_element_type=jnp.float32)
    pos = posx + carry[...]
    p1 = jnp.sum(jnp.where(lane == i1, pos, 0.0), axis=1, keepdims=True)
    p2 = jnp.sum(jnp.where(lane == i2, pos, 0.0), axis=1, keepdims=True)
    carry[...] += jnp.sum(r.astype(jnp.float32), axis=0, keepdims=True)

    e1_ref[...] = i1
    e2_ref[...] = i2
    g1_ref[...] = M1
    g2_ref[...] = M2
    p1_ref[...] = p1.astype(jnp.int32)
    p2_ref[...] = p2.astype(jnp.int32)
    cnt_ref[...] = carry[...].astype(jnp.int32)

    t = tmp[...]
    mu = jnp.sum(t) / E
    var = jnp.sum((t - mu) ** 2) / (E - 1)
    loss_ref[...] = jnp.full((1, 1), var / (mu * mu), jnp.float32)


def _gating(x, gwt, nwt, eps):
    tok_i = pl.BlockSpec((GB, 1), lambda i: (i, 0))
    return pl.pallas_call(
        _gating_body,
        grid=(NGB,),
        in_specs=[
            pl.BlockSpec((GB, D), lambda i: (i, 0)),
            pl.BlockSpec((D, E), lambda i: (0, 0)),
            pl.BlockSpec((D, 1), lambda i: (0, 0)),
            pl.BlockSpec((GB, E), lambda i: (i, 0)),
        ],
        out_specs=[tok_i, tok_i, tok_i, tok_i, tok_i, tok_i,
                   pl.BlockSpec((1, E), lambda i: (0, 0)),
                   pl.BlockSpec((1, 1), lambda i: (0, 0))],
        out_shape=[
            jax.ShapeDtypeStruct((N, 1), jnp.int32),
            jax.ShapeDtypeStruct((N, 1), jnp.int32),
            jax.ShapeDtypeStruct((N, 1), jnp.float32),
            jax.ShapeDtypeStruct((N, 1), jnp.float32),
            jax.ShapeDtypeStruct((N, 1), jnp.int32),
            jax.ShapeDtypeStruct((N, 1), jnp.int32),
            jax.ShapeDtypeStruct((1, E), jnp.int32),
            jax.ShapeDtypeStruct((1, 1), jnp.float32),
        ],
        scratch_shapes=[pltpu.VMEM((1, E), jnp.float32),
                        pltpu.VMEM((1, E), jnp.float32)],
    )(x, gwt, nwt, eps)


def _dispatch_sc(x, s1, s2, g1w, g2w):
    mesh = plsc.VectorSubcoreMesh(core_axis_name="c", subcore_axis_name="s")

    @functools.partial(
        pl.kernel,
        out_type=(jax.ShapeDtypeStruct((S, D), jnp.float32),
                  jax.ShapeDtypeStruct((S, 128), jnp.float32)),
        mesh=mesh,
        scratch_types=[
            pltpu.VMEM((TPW,), jnp.int32),
            pltpu.VMEM((TPW,), jnp.int32),
            pltpu.VMEM((TPW, D), jnp.float32),
            pltpu.VMEM((TPW, 128), jnp.float32),
            pltpu.VMEM((TPW, 128), jnp.float32),
            pltpu.SemaphoreType.DMA,
        ],
    )
    def k(x_hbm, s1_hbm, s2_hbm, g1w_hbm, g2w_hbm, xs_hbm, wg_hbm,
          i1v, i2v, xv, g1v, g2v, sem):
        wid = lax.axis_index("s") * 2 + lax.axis_index("c")
        base = wid * TPW
        c1 = pltpu.async_copy(s1_hbm.at[pl.ds(base, TPW)], i1v, sem)
        c2 = pltpu.async_copy(s2_hbm.at[pl.ds(base, TPW)], i2v, sem)
        c3 = pltpu.async_copy(x_hbm.at[pl.ds(base, TPW)], xv, sem)
        c4 = pltpu.async_copy(g1w_hbm.at[pl.ds(base, TPW)], g1v, sem)
        c5 = pltpu.async_copy(g2w_hbm.at[pl.ds(base, TPW)], g2v, sem)
        for c in (c1, c2, c3, c4, c5):
            c.wait()
        d1 = pltpu.async_copy(xv, xs_hbm.at[i1v], sem)
        d2 = pltpu.async_copy(xv, xs_hbm.at[i2v], sem)
        d3 = pltpu.async_copy(g1v, wg_hbm.at[i1v], sem)
        d4 = pltpu.async_copy(g2v, wg_hbm.at[i2v], sem)
        for d in (d1, d2, d3, d4):
            d.wait()

    return k(x, s1, s2, g1w, g2w)


def _ffn_body(be_ref, xs_ref, w1_ref, b1_ref, w2_ref, b2_ref, wg_ref, ys_ref):
    kt = (((1,), (1,)), ((), ()))
    xb = xs_ref[...].astype(jnp.bfloat16)
    w1b = w1_ref[0].astype(jnp.bfloat16)
    h = lax.dot_general(xb, w1b, kt, preferred_element_type=jnp.float32)
    h = jnp.maximum(h + b1_ref[0], 0.0).astype(jnp.bfloat16)
    w2b = w2_ref[0].astype(jnp.bfloat16)
    y = lax.dot_general(h, w2b, kt, preferred_element_type=jnp.float32)
    ys_ref[...] = (y + b2_ref[0]) * wg_ref[:, :1]


def _ffn(be, xs, w1t, b1, w2t, b2, wg):
    grid_spec = pltpu.PrefetchScalarGridSpec(
        num_scalar_prefetch=1,
        grid=(NBLK,),
        in_specs=[
            pl.BlockSpec((B, D), lambda b, be: (b, 0)),
            pl.BlockSpec((1, H, D), lambda b, be: (be[b], 0, 0)),
            pl.BlockSpec((1, 1, H), lambda b, be: (be[b], 0, 0)),
            pl.BlockSpec((1, O, H), lambda b, be: (be[b], 0, 0)),
            pl.BlockSpec((1, 1, O), lambda b, be: (be[b], 0, 0)),
            pl.BlockSpec((B, 128), lambda b, be: (b, 0)),
        ],
        out_specs=pl.BlockSpec((B, O), lambda b, be: (b, 0)),
    )
    return pl.pallas_call(
        _ffn_body,
        grid_spec=grid_spec,
        out_shape=jax.ShapeDtypeStruct((S, O), jnp.float32),
    )(be, xs, w1t, b1, w2t, b2, wg)


def _combine_sc(ys, s1, s2):
    mesh = plsc.VectorSubcoreMesh(core_axis_name="c", subcore_axis_name="s")

    @functools.partial(
        pl.kernel,
        out_type=jax.ShapeDtypeStruct((N, O), jnp.float32),
        mesh=mesh,
        scratch_types=[
            pltpu.VMEM((TPW,), jnp.int32),
            pltpu.VMEM((TPW,), jnp.int32),
            pltpu.VMEM((TPW, O), jnp.float32),
            pltpu.VMEM((TPW, O), jnp.float32),
            pltpu.SemaphoreType.DMA,
        ],
    )
    def k(ys_hbm, s1_hbm, s2_hbm, out_hbm, i1v, i2v, av, bv, sem):
        wid = lax.axis_index("s") * 2 + lax.axis_index("c")
        base = wid * TPW
        c1 = pltpu.async_copy(s1_hbm.at[pl.ds(base, TPW)], i1v, sem)
        c2 = pltpu.async_copy(s2_hbm.at[pl.ds(base, TPW)], i2v, sem)
        c1.wait()
        c2.wait()
        g1 = pltpu.async_copy(ys_hbm.at[i1v], av, sem)
        g2 = pltpu.async_copy(ys_hbm.at[i2v], bv, sem)
        g1.wait()
        g2.wait()

        @pl.loop(0, TPW)
        def _(j):
            @pl.loop(0, O, step=16)
            def _(c):
                av.at[j, pl.ds(c, 16)][...] = (av[j, pl.ds(c, 16)]
                                               + bv[j, pl.ds(c, 16)])

        pltpu.sync_copy(av, out_hbm.at[pl.ds(base, TPW)])

    return k(ys, s1, s2)


def _finalize_body(e1_ref, e2_ref, p1_ref, p2_ref, g1_ref, g2_ref, cnt_ref,
                   s1_ref, s2_ref, g1w_ref, g2w_ref, be_ref):
    nblk = (cnt_ref[...] + (B - 1)) >> 8
    nblk8 = jnp.broadcast_to(nblk, (E, E))
    re8 = lax.broadcasted_iota(jnp.int32, (E, E), 0)
    ce8 = lax.broadcasted_iota(jnp.int32, (E, E), 1)
    starts = jnp.sum(jnp.where(ce8 < re8, nblk8, 0), axis=1, keepdims=True)
    off = starts * B

    bid = lax.broadcasted_iota(jnp.int32, (NBLK, E), 0)
    st_b = jnp.broadcast_to(starts.reshape(1, E), (NBLK, E))
    be = jnp.sum((bid >= st_b).astype(jnp.int32), axis=1, keepdims=True) - 1
    be_ref[...] = jnp.clip(be, 0, E - 1)

    laneE = lax.broadcasted_iota(jnp.int32, (N, E), 1)
    offrow = jnp.broadcast_to(off.reshape(1, E), (N, E))
    o1 = jnp.sum(jnp.where(laneE == e1_ref[...], offrow, 0), axis=1,
                 keepdims=True)
    o2 = jnp.sum(jnp.where(laneE == e2_ref[...], offrow, 0), axis=1,
                 keepdims=True)
    s1_ref[...] = o1 + p1_ref[...]
    s2_ref[...] = o2 + p2_ref[...]
    g1w_ref[...] = jnp.broadcast_to(g1_ref[...], (N, 128))
    g2w_ref[...] = jnp.broadcast_to(g2_ref[...], (N, 128))


def _finalize(e1, e2, p1, p2, g1, g2, counts):
    full = lambda shape: pl.BlockSpec(shape, lambda: tuple(0 for _ in shape))
    return pl.pallas_call(
        _finalize_body,
        in_specs=[full((N, 1))] * 6 + [full((1, E))],
        out_specs=[full((N, 1)), full((N, 1)), full((N, 128)),
                   full((N, 128)), full((NBLK, 1))],
        out_shape=[
            jax.ShapeDtypeStruct((N, 1), jnp.int32),
            jax.ShapeDtypeStruct((N, 1), jnp.int32),
            jax.ShapeDtypeStruct((N, 128), jnp.float32),
            jax.ShapeDtypeStruct((N, 128), jnp.float32),
            jax.ShapeDtypeStruct((NBLK, 1), jnp.int32),
        ],
    )(e1, e2, p1, p2, g1, g2, counts)


def kernel(x, gate_w, noise_w, w1, b1, w2, b2, noise_eps):
    gwt = gate_w.T.astype(jnp.bfloat16)
    nwt = noise_w.T.astype(jnp.bfloat16)

    e1, e2, g1, g2, p1, p2, counts, loss = _gating(x, gwt, nwt, noise_eps)
    s1_2d, s2_2d, g1w, g2w, be_2d = _finalize(e1, e2, p1, p2, g1, g2, counts)
    s1 = s1_2d.reshape(N)
    s2 = s2_2d.reshape(N)
    be = be_2d.reshape(NBLK)

    xs, wg = _dispatch_sc(x, s1, s2, g1w, g2w)
    ys = _ffn(be, xs, w1, b1.reshape(E, 1, H), w2, b2.reshape(E, 1, O), wg)
    out = _combine_sc(ys, s1, s2)
    return out, loss[0, 0]


# speedup vs baseline: 1.7409x; 1.1345x over previous
import functools

import jax
import jax.numpy as jnp
from jax import lax
from jax.experimental import pallas as pl
from jax.experimental.pallas import tpu as pltpu
from jax.experimental.pallas import tpu_sc as plsc

N = 2048
D = 768
H = 3072
O = 768
E = 8
GB = 256
NGB = N // GB
B = 256
NBLK = 24
S = NBLK * B
NW = 32
TPW = N // NW


def _gating_body(x_ref, gw9_ref, eps_ref,
                 e1_ref, e2_ref, g1_ref, g2_ref, p1_ref, p2_ref,
                 cnt_ref, loss_ref, carry, tmp):
    i = pl.program_id(0)

    @pl.when(i == 0)
    def _():
        carry[...] = jnp.zeros_like(carry)
        tmp[...] = jnp.zeros_like(tmp)

    kt = (((1,), (1,)), ((), ()))
    xb = x_ref[...].astype(jnp.bfloat16)
    gn = lax.dot_general(xb, gw9_ref[...].astype(jnp.bfloat16), kt,
                         preferred_element_type=jnp.float32)
    g = gn[:, :E]
    nz = gn[:, E:E + 1]
    sp = jnp.maximum(nz, 0.0) + jnp.log(1.0 + jnp.exp(-jnp.abs(nz)))
    h = g + eps_ref[...] * sp

    neg_inf = jnp.float32(-jnp.inf)
    m1 = jnp.min(h, axis=1, keepdims=True)
    m2 = jnp.min(jnp.where(h == m1, jnp.float32(jnp.inf), h), axis=1,
                 keepdims=True)
    h = jnp.where((h == m1) | (h == m2), neg_inf, h)

    mx = jnp.max(h, axis=1, keepdims=True)
    p = jnp.exp(h - mx)
    L = p / jnp.sum(p, axis=1, keepdims=True)
    tmp[...] += jnp.sum(L, axis=0, keepdims=True)

    lane = lax.broadcasted_iota(jnp.int32, (GB, E), 1)
    M1 = jnp.max(L, axis=1, keepdims=True)
    i1 = jnp.min(jnp.where(L == M1, lane, E), axis=1, keepdims=True)
    L2 = jnp.where(lane == i1, -1.0, L)
    M2 = jnp.max(L2, axis=1, keepdims=True)
    i2 = jnp.min(jnp.where(L2 == M2, lane, E), axis=1, keepdims=True)

    r = (jnp.equal(lane, i1) | jnp.equal(lane, i2)).astype(jnp.bfloat16)
    row = lax.broadcasted_iota(jnp.int32, (GB, GB), 0)
    col = lax.broadcasted_iota(jnp.int32, (GB, GB), 1)
    tri = (row > col).astype(jnp.bfloat16)
    posx = jnp.dot(tri, r, preferred_element_type=jnp.float32)
    pos = posx + carry[...]
    p1 = jnp.sum(jnp.where(lane == i1, pos, 0.0), axis=1, keepdims=True)
    p2 = jnp.sum(jnp.where(lane == i2, pos, 0.0), axis=1, keepdims=True)
    carry[...] += jnp.sum(r.astype(jnp.float32), axis=0, keepdims=True)

    e1_ref[...] = i1
    e2_ref[...] = i2
    g1_ref[...] = M1
    g2_ref[...] = M2
    p1_ref[...] = p1.astype(jnp.int32)
    p2_ref[...] = p2.astype(jnp.int32)
    cnt_ref[...] = carry[...].astype(jnp.int32)

    t = tmp[...]
    mu = jnp.sum(t) / E
    var = jnp.sum((t - mu) ** 2) / (E - 1)
    loss_ref[...] = jnp.full((1, 1), var / (mu * mu), jnp.float32)


def _gating(x, gw9, eps):
    tok_i = pl.BlockSpec((GB, 1), lambda i: (i, 0))
    return pl.pallas_call(
        _gating_body,
        grid=(NGB,),
        in_specs=[
            pl.BlockSpec((GB, D), lambda i: (i, 0)),
            pl.BlockSpec((E + 1, D), lambda i: (0, 0)),
            pl.BlockSpec((GB, E), lambda i: (i, 0)),
        ],
        out_specs=[tok_i, tok_i, tok_i, tok_i, tok_i, tok_i,
                   pl.BlockSpec((1, E), lambda i: (0, 0)),
                   pl.BlockSpec((1, 1), lambda i: (0, 0))],
        out_shape=[
            jax.ShapeDtypeStruct((N, 1), jnp.int32),
            jax.ShapeDtypeStruct((N, 1), jnp.int32),
            jax.ShapeDtypeStruct((N, 1), jnp.float32),
            jax.ShapeDtypeStruct((N, 1), jnp.float32),
            jax.ShapeDtypeStruct((N, 1), jnp.int32),
            jax.ShapeDtypeStruct((N, 1), jnp.int32),
            jax.ShapeDtypeStruct((1, E), jnp.int32),
            jax.ShapeDtypeStruct((1, 1), jnp.float32),
        ],
        scratch_shapes=[pltpu.VMEM((1, E), jnp.float32),
                        pltpu.VMEM((1, E), jnp.float32)],
    )(x, gw9, eps)


def _dispatch_sc(x, s1, s2, g1w, g2w):
    mesh = plsc.VectorSubcoreMesh(core_axis_name="c", subcore_axis_name="s")

    @functools.partial(
        pl.kernel,
        out_type=(jax.ShapeDtypeStruct((S, D), jnp.float32),
                  jax.ShapeDtypeStruct((S, 128), jnp.float32)),
        mesh=mesh,
        scratch_types=[
            pltpu.VMEM((TPW,), jnp.int32),
            pltpu.VMEM((TPW,), jnp.int32),
            pltpu.VMEM((TPW, D), jnp.float32),
            pltpu.VMEM((TPW, 128), jnp.float32),
            pltpu.VMEM((TPW, 128), jnp.float32),
            pltpu.SemaphoreType.DMA,
        ],
    )
    def k(x_hbm, s1_hbm, s2_hbm, g1w_hbm, g2w_hbm, xs_hbm, wg_hbm,
          i1v, i2v, xv, g1v, g2v, sem):
        wid = lax.axis_index("s") * 2 + lax.axis_index("c")
        base = wid * TPW
        c1 = pltpu.async_copy(s1_hbm.at[pl.ds(base, TPW)], i1v, sem)
        c2 = pltpu.async_copy(s2_hbm.at[pl.ds(base, TPW)], i2v, sem)
        c3 = pltpu.async_copy(x_hbm.at[pl.ds(base, TPW)], xv, sem)
        c4 = pltpu.async_copy(g1w_hbm.at[pl.ds(base, TPW)], g1v, sem)
        c5 = pltpu.async_copy(g2w_hbm.at[pl.ds(base, TPW)], g2v, sem)
        for c in (c1, c2, c3, c4, c5):
            c.wait()
        d1 = pltpu.async_copy(xv, xs_hbm.at[i1v], sem)
        d2 = pltpu.async_copy(xv, xs_hbm.at[i2v], sem)
        d3 = pltpu.async_copy(g1v, wg_hbm.at[i1v], sem)
        d4 = pltpu.async_copy(g2v, wg_hbm.at[i2v], sem)
        for d in (d1, d2, d3, d4):
            d.wait()

    return k(x, s1, s2, g1w, g2w)


def _ffn_body(be_ref, xs_ref, w1_ref, b1_ref, w2_ref, b2_ref, wg_ref, ys_ref):
    b = pl.program_id(0)

    @pl.when(be_ref[b] < E)
    def _():
        kt = (((1,), (1,)), ((), ()))
        xb = xs_ref[...].astype(jnp.bfloat16)
        w1b = w1_ref[0].astype(jnp.bfloat16)
        h = lax.dot_general(xb, w1b, kt, preferred_element_type=jnp.float32)
        h = jnp.maximum(h + b1_ref[0], 0.0).astype(jnp.bfloat16)
        w2b = w2_ref[0].astype(jnp.bfloat16)
        y = lax.dot_general(h, w2b, kt, preferred_element_type=jnp.float32)
        ys_ref[...] = (y + b2_ref[0]) * wg_ref[:, :1]


def _ffn(be, xs, w1t, b1, w2t, b2, wg):
    grid_spec = pltpu.PrefetchScalarGridSpec(
        num_scalar_prefetch=1,
        grid=(NBLK,),
        in_specs=[
            pl.BlockSpec((B, D), lambda b, be: (b, 0)),
            pl.BlockSpec((1, H, D), lambda b, be: (be[b] & 7, 0, 0)),
            pl.BlockSpec((1, 1, H), lambda b, be: (be[b] & 7, 0, 0)),
            pl.BlockSpec((1, O, H), lambda b, be: (be[b] & 7, 0, 0)),
            pl.BlockSpec((1, 1, O), lambda b, be: (be[b] & 7, 0, 0)),
            pl.BlockSpec((B, 128), lambda b, be: (b, 0)),
        ],
        out_specs=pl.BlockSpec((B, O), lambda b, be: (b, 0)),
    )
    return pl.pallas_call(
        _ffn_body,
        grid_spec=grid_spec,
        out_shape=jax.ShapeDtypeStruct((S, O), jnp.float32),
    )(be, xs, w1t, b1, w2t, b2, wg)


def _combine_sc(ys, s1, s2):
    mesh = plsc.VectorSubcoreMesh(core_axis_name="c", subcore_axis_name="s")

    @functools.partial(
        pl.kernel,
        out_type=jax.ShapeDtypeStruct((N, O), jnp.float32),
        mesh=mesh,
        scratch_types=[
            pltpu.VMEM((TPW,), jnp.int32),
            pltpu.VMEM((TPW,), jnp.int32),
            pltpu.VMEM((TPW, O), jnp.float32),
            pltpu.VMEM((TPW, O), jnp.float32),
            pltpu.SemaphoreType.DMA,
        ],
    )
    def k(ys_hbm, s1_hbm, s2_hbm, out_hbm, i1v, i2v, av, bv, sem):
        wid = lax.axis_index("s") * 2 + lax.axis_index("c")
        base = wid * TPW
        c1 = pltpu.async_copy(s1_hbm.at[pl.ds(base, TPW)], i1v, sem)
        c2 = pltpu.async_copy(s2_hbm.at[pl.ds(base, TPW)], i2v, sem)
        c1.wait()
        c2.wait()
        g1 = pltpu.async_copy(ys_hbm.at[i1v], av, sem)
        g2 = pltpu.async_copy(ys_hbm.at[i2v], bv, sem)
        g1.wait()
        g2.wait()

        @pl.loop(0, TPW)
        def _(j):
            for c in range(0, O, 16):
                av.at[j, pl.ds(c, 16)][...] = (av[j, pl.ds(c, 16)]
                                               + bv[j, pl.ds(c, 16)])

        pltpu.sync_copy(av, out_hbm.at[pl.ds(base, TPW)])

    return k(ys, s1, s2)


def _finalize_body(e1_ref, e2_ref, p1_ref, p2_ref, g1_ref, g2_ref, cnt_ref,
                   s1_ref, s2_ref, g1w_ref, g2w_ref, be_ref):
    nblk = (cnt_ref[...] + (B - 1)) >> 8
    nblk8 = jnp.broadcast_to(nblk, (E, E))
    re8 = lax.broadcasted_iota(jnp.int32, (E, E), 0)
    ce8 = lax.broadcasted_iota(jnp.int32, (E, E), 1)
    starts = jnp.sum(jnp.where(ce8 < re8, nblk8, 0), axis=1, keepdims=True)
    off = starts * B

    bid = lax.broadcasted_iota(jnp.int32, (NBLK, E), 0)
    st_b = jnp.broadcast_to(starts.reshape(1, E), (NBLK, E))
    be = jnp.sum((bid >= st_b).astype(jnp.int32), axis=1, keepdims=True) - 1
    used = jnp.sum(nblk)
    bcol = lax.broadcasted_iota(jnp.int32, (NBLK, 1), 0)
    be_ref[...] = jnp.clip(be, 0, E - 1) + 8 * (bcol >= used).astype(jnp.int32)

    laneE = lax.broadcasted_iota(jnp.int32, (N, E), 1)
    offrow = jnp.broadcast_to(off.reshape(1, E), (N, E))
    o1 = jnp.sum(jnp.where(laneE == e1_ref[...], offrow, 0), axis=1,
                 keepdims=True)
    o2 = jnp.sum(jnp.where(laneE == e2_ref[...], offrow, 0), axis=1,
                 keepdims=True)
    s1_ref[...] = o1 + p1_ref[...]
    s2_ref[...] = o2 + p2_ref[...]
    g1w_ref[...] = jnp.broadcast_to(g1_ref[...], (N, 128))
    g2w_ref[...] = jnp.broadcast_to(g2_ref[...], (N, 128))


def _finalize(e1, e2, p1, p2, g1, g2, counts):
    full = lambda shape: pl.BlockSpec(shape, lambda: tuple(0 for _ in shape))
    return pl.pallas_call(
        _finalize_body,
        in_specs=[full((N, 1))] * 6 + [full((1, E))],
        out_specs=[full((N, 1)), full((N, 1)), full((N, 128)),
                   full((N, 128)), full((NBLK, 1))],
        out_shape=[
            jax.ShapeDtypeStruct((N, 1), jnp.int32),
            jax.ShapeDtypeStruct((N, 1), jnp.int32),
            jax.ShapeDtypeStruct((N, 128), jnp.float32),
            jax.ShapeDtypeStruct((N, 128), jnp.float32),
            jax.ShapeDtypeStruct((NBLK, 1), jnp.int32),
        ],
    )(e1, e2, p1, p2, g1, g2, counts)


def kernel(x, gate_w, noise_w, w1, b1, w2, b2, noise_eps):
    gw9 = jnp.concatenate([gate_w, noise_w], axis=0)
    e1, e2, g1, g2, p1, p2, counts, loss = _gating(x, gw9, noise_eps)
    s1_2d, s2_2d, g1w, g2w, be_2d = _finalize(e1, e2, p1, p2, g1, g2, counts)
    s1 = s1_2d.reshape(N)
    s2 = s2_2d.reshape(N)
    be = be_2d.reshape(NBLK)

    xs, wg = _dispatch_sc(x, s1, s2, g1w, g2w)
    ys = _ffn(be, xs, w1, b1.reshape(E, 1, H), w2, b2.reshape(E, 1, O), wg)
    out = _combine_sc(ys, s1, s2)
    return out, loss[0, 0]


# speedup vs baseline: 1.7678x; 1.0154x over previous
import functools

import jax
import jax.numpy as jnp
from jax import lax
from jax.experimental import pallas as pl
from jax.experimental.pallas import tpu as pltpu
from jax.experimental.pallas import tpu_sc as plsc

N = 2048
D = 768
H = 3072
O = 768
E = 8
GB = 256
NGB = N // GB
B = 256
NBLK = 24
S = NBLK * B
NW = 32
TPW = N // NW


def _gating_body(x_ref, gw9_ref, eps_ref,
                 e1_ref, e2_ref, g1_ref, g2_ref, p1_ref, p2_ref,
                 cnt_ref, loss_ref, carry, tmp):
    i = pl.program_id(0)

    @pl.when(i == 0)
    def _():
        carry[...] = jnp.zeros_like(carry)
        tmp[...] = jnp.zeros_like(tmp)

    kt = (((1,), (1,)), ((), ()))
    xb = x_ref[...].astype(jnp.bfloat16)
    gn = lax.dot_general(xb, gw9_ref[...].astype(jnp.bfloat16), kt,
                         preferred_element_type=jnp.float32)
    g = gn[:, :E]
    nz = gn[:, E:E + 1]
    sp = jnp.maximum(nz, 0.0) + jnp.log(1.0 + jnp.exp(-jnp.abs(nz)))
    h = g + eps_ref[...] * sp

    neg_inf = jnp.float32(-jnp.inf)
    m1 = jnp.min(h, axis=1, keepdims=True)
    m2 = jnp.min(jnp.where(h == m1, jnp.float32(jnp.inf), h), axis=1,
                 keepdims=True)
    h = jnp.where((h == m1) | (h == m2), neg_inf, h)

    mx = jnp.max(h, axis=1, keepdims=True)
    p = jnp.exp(h - mx)
    L = p / jnp.sum(p, axis=1, keepdims=True)
    tmp[...] += jnp.sum(L, axis=0, keepdims=True)

    lane = lax.broadcasted_iota(jnp.int32, (GB, E), 1)
    M1 = jnp.max(L, axis=1, keepdims=True)
    i1 = jnp.min(jnp.where(L == M1, lane, E), axis=1, keepdims=True)
    L2 = jnp.where(lane == i1, -1.0, L)
    M2 = jnp.max(L2, axis=1, keepdims=True)
    i2 = jnp.min(jnp.where(L2 == M2, lane, E), axis=1, keepdims=True)

    r = (jnp.equal(lane, i1) | jnp.equal(lane, i2)).astype(jnp.bfloat16)
    row = lax.broadcasted_iota(jnp.int32, (GB, GB), 0)
    col = lax.broadcasted_iota(jnp.int32, (GB, GB), 1)
    tri = (row > col).astype(jnp.bfloat16)
    posx = jnp.dot(tri, r, preferred_element_type=jnp.float32)
    pos = posx + carry[...]
    p1 = jnp.sum(jnp.where(lane == i1, pos, 0.0), axis=1, keepdims=True)
    p2 = jnp.sum(jnp.where(lane == i2, pos, 0.0), axis=1, keepdims=True)
    carry[...] += jnp.sum(r.astype(jnp.float32), axis=0, keepdims=True)

    e1_ref[...] = i1
    e2_ref[...] = i2
    g1_ref[...] = M1
    g2_ref[...] = M2
    p1_ref[...] = p1.astype(jnp.int32)
    p2_ref[...] = p2.astype(jnp.int32)
    cnt_ref[...] = carry[...].astype(jnp.int32)

    t = tmp[...]
    mu = jnp.sum(t) / E
    var = jnp.sum((t - mu) ** 2) / (E - 1)
    loss_ref[...] = jnp.full((1, 1), var / (mu * mu), jnp.float32)


def _gating(x, gw9, eps):
    tok_i = pl.BlockSpec((GB, 1), lambda i: (i, 0))
    return pl.pallas_call(
        _gating_body,
        grid=(NGB,),
        in_specs=[
            pl.BlockSpec((GB, D), lambda i: (i, 0)),
            pl.BlockSpec((E + 1, D), lambda i: (0, 0)),
            pl.BlockSpec((GB, E), lambda i: (i, 0)),
        ],
        out_specs=[tok_i, tok_i, tok_i, tok_i, tok_i, tok_i,
                   pl.BlockSpec((1, E), lambda i: (0, 0)),
                   pl.BlockSpec((1, 1), lambda i: (0, 0))],
        out_shape=[
            jax.ShapeDtypeStruct((N, 1), jnp.int32),
            jax.ShapeDtypeStruct((N, 1), jnp.int32),
            jax.ShapeDtypeStruct((N, 1), jnp.float32),
            jax.ShapeDtypeStruct((N, 1), jnp.float32),
            jax.ShapeDtypeStruct((N, 1), jnp.int32),
            jax.ShapeDtypeStruct((N, 1), jnp.int32),
            jax.ShapeDtypeStruct((1, E), jnp.int32),
            jax.ShapeDtypeStruct((1, 1), jnp.float32),
        ],
        scratch_shapes=[pltpu.VMEM((1, E), jnp.float32),
                        pltpu.VMEM((1, E), jnp.float32)],
    )(x, gw9, eps)


def _dispatch_sc(x, s1, s2, g1w, g2w):
    mesh = plsc.VectorSubcoreMesh(core_axis_name="c", subcore_axis_name="s")

    @functools.partial(
        pl.kernel,
        out_type=(jax.ShapeDtypeStruct((S, D), jnp.float32),
                  jax.ShapeDtypeStruct((S, 128), jnp.float32)),
        mesh=mesh,
        scratch_types=[
            pltpu.VMEM((TPW,), jnp.int32),
            pltpu.VMEM((TPW,), jnp.int32),
            pltpu.VMEM((TPW, D), jnp.float32),
            pltpu.VMEM((TPW, 128), jnp.float32),
            pltpu.VMEM((TPW, 128), jnp.float32),
            pltpu.SemaphoreType.DMA,
        ],
    )
    def k(x_hbm, s1_hbm, s2_hbm, g1w_hbm, g2w_hbm, xs_hbm, wg_hbm,
          i1v, i2v, xv, g1v, g2v, sem):
        wid = lax.axis_index("s") * 2 + lax.axis_index("c")
        base = wid * TPW
        c1 = pltpu.async_copy(s1_hbm.at[pl.ds(base, TPW)], i1v, sem)
        c2 = pltpu.async_copy(s2_hbm.at[pl.ds(base, TPW)], i2v, sem)
        c3 = pltpu.async_copy(x_hbm.at[pl.ds(base, TPW)], xv, sem)
        c4 = pltpu.async_copy(g1w_hbm.at[pl.ds(base, TPW)], g1v, sem)
        c5 = pltpu.async_copy(g2w_hbm.at[pl.ds(base, TPW)], g2v, sem)
        for c in (c1, c2, c3, c4, c5):
            c.wait()
        d1 = pltpu.async_copy(xv, xs_hbm.at[i1v], sem)
        d2 = pltpu.async_copy(xv, xs_hbm.at[i2v], sem)
        d3 = pltpu.async_copy(g1v, wg_hbm.at[i1v], sem)
        d4 = pltpu.async_copy(g2v, wg_hbm.at[i2v], sem)
        for d in (d1, d2, d3, d4):
            d.wait()

    return k(x, s1, s2, g1w, g2w)


def _ffn_body(be_ref, xs_ref, w1_ref, b1_ref, w2_ref, b2_ref, wg_ref, ys_ref):
    b = pl.program_id(0)

    @pl.when(be_ref[b] < E)
    def _():
        kt = (((1,), (1,)), ((), ()))
        xb = xs_ref[...].astype(jnp.bfloat16)
        w1b = w1_ref[0].astype(jnp.bfloat16)
        h = lax.dot_general(xb, w1b, kt, preferred_element_type=jnp.float32)
        h = jnp.maximum(h + b1_ref[0], 0.0).astype(jnp.bfloat16)
        w2b = w2_ref[0].astype(jnp.bfloat16)
        y = lax.dot_general(h, w2b, kt, preferred_element_type=jnp.float32)
        ys_ref[...] = (y + b2_ref[0]) * wg_ref[:, :1]


def _ffn(be, xs, w1t, b1, w2t, b2, wg):
    grid_spec = pltpu.PrefetchScalarGridSpec(
        num_scalar_prefetch=1,
        grid=(NBLK,),
        in_specs=[
            pl.BlockSpec((B, D), lambda b, be: (b, 0)),
            pl.BlockSpec((1, H, D), lambda b, be: (be[b] & 7, 0, 0)),
            pl.BlockSpec((1, 1, H), lambda b, be: (be[b] & 7, 0, 0)),
            pl.BlockSpec((1, O, H), lambda b, be: (be[b] & 7, 0, 0)),
            pl.BlockSpec((1, 1, O), lambda b, be: (be[b] & 7, 0, 0)),
            pl.BlockSpec((B, 128), lambda b, be: (b, 0)),
        ],
        out_specs=pl.BlockSpec((B, O), lambda b, be: (b, 0)),
    )
    return pl.pallas_call(
        _ffn_body,
        grid_spec=grid_spec,
        out_shape=jax.ShapeDtypeStruct((S, O), jnp.float32),
    )(be, xs, w1t, b1, w2t, b2, wg)


def _combine_sc(ys, s1, s2):
    mesh = plsc.VectorSubcoreMesh(core_axis_name="c", subcore_axis_name="s")

    @functools.partial(
        pl.kernel,
        out_type=jax.ShapeDtypeStruct((N, O), jnp.float32),
        mesh=mesh,
        scratch_types=[
            pltpu.VMEM((TPW,), jnp.int32),
            pltpu.VMEM((TPW,), jnp.int32),
            pltpu.VMEM((TPW, O), jnp.float32),
            pltpu.VMEM((TPW, O), jnp.float32),
            pltpu.SemaphoreType.DMA,
        ],
    )
    def k(ys_hbm, s1_hbm, s2_hbm, out_hbm, i1v, i2v, av, bv, sem):
        wid = lax.axis_index("s") * 2 + lax.axis_index("c")
        base = wid * TPW
        c1 = pltpu.async_copy(s1_hbm.at[pl.ds(base, TPW)], i1v, sem)
        c2 = pltpu.async_copy(s2_hbm.at[pl.ds(base, TPW)], i2v, sem)
        c1.wait()
        c2.wait()
        g1 = pltpu.async_copy(ys_hbm.at[i1v], av, sem)
        g2 = pltpu.async_copy(ys_hbm.at[i2v], bv, sem)
        g1.wait()
        g2.wait()

        @pl.loop(0, TPW)
        def _(j):
            for c in range(0, O, 16):
                av.at[j, pl.ds(c, 16)][...] = (av[j, pl.ds(c, 16)]
                                               + bv[j, pl.ds(c, 16)])

        pltpu.sync_copy(av, out_hbm.at[pl.ds(base, TPW)])

    return k(ys, s1, s2)


def _finalize_body(e1_ref, e2_ref, p1_ref, p2_ref, g1_ref, g2_ref, cnt_ref,
                   s1_ref, s2_ref, g1w_ref, g2w_ref, be_ref):
    nblk = (cnt_ref[...] + (B - 1)) >> 8
    nblk8 = jnp.broadcast_to(nblk, (E, E))
    re8 = lax.broadcasted_iota(jnp.int32, (E, E), 0)
    ce8 = lax.broadcasted_iota(jnp.int32, (E, E), 1)
    starts = jnp.sum(jnp.where(ce8 < re8, nblk8, 0), axis=1, keepdims=True)
    off = starts * B

    bid = lax.broadcasted_iota(jnp.int32, (NBLK, E), 0)
    st_b = jnp.broadcast_to(starts.reshape(1, E), (NBLK, E))
    be = jnp.sum((bid >= st_b).astype(jnp.int32), axis=1, keepdims=True) - 1
    used = jnp.sum(nblk)
    bcol = lax.broadcasted_iota(jnp.int32, (NBLK, 1), 0)
    be_ref[...] = (jnp.clip(be, 0, E - 1)
                   + 8 * (bcol >= used).astype(jnp.int32)).reshape(NBLK)

    laneE = lax.broadcasted_iota(jnp.int32, (N, E), 1)
    offrow = jnp.broadcast_to(off.reshape(1, E), (N, E))
    o1 = jnp.sum(jnp.where(laneE == e1_ref[...], offrow, 0), axis=1,
                 keepdims=True)
    o2 = jnp.sum(jnp.where(laneE == e2_ref[...], offrow, 0), axis=1,
                 keepdims=True)
    s1_ref[...] = (o1 + p1_ref[...]).reshape(N)
    s2_ref[...] = (o2 + p2_ref[...]).reshape(N)
    g1w_ref[...] = jnp.broadcast_to(g1_ref[...], (N, 128))
    g2w_ref[...] = jnp.broadcast_to(g2_ref[...], (N, 128))


def _finalize(e1, e2, p1, p2, g1, g2, counts):
    full = lambda shape: pl.BlockSpec(shape, lambda: tuple(0 for _ in shape))
    return pl.pallas_call(
        _finalize_body,
        in_specs=[full((N, 1))] * 6 + [full((1, E))],
        out_specs=[full((N,)), full((N,)), full((N, 128)),
                   full((N, 128)), full((NBLK,))],
        out_shape=[
            jax.ShapeDtypeStruct((N,), jnp.int32),
            jax.ShapeDtypeStruct((N,), jnp.int32),
            jax.ShapeDtypeStruct((N, 128), jnp.float32),
            jax.ShapeDtypeStruct((N, 128), jnp.float32),
            jax.ShapeDtypeStruct((NBLK,), jnp.int32),
        ],
    )(e1, e2, p1, p2, g1, g2, counts)


def kernel(x, gate_w, noise_w, w1, b1, w2, b2, noise_eps):
    gw9 = jnp.concatenate([gate_w, noise_w], axis=0)
    e1, e2, g1, g2, p1, p2, counts, loss = _gating(x, gw9, noise_eps)
    s1, s2, g1w, g2w, be = _finalize(e1, e2, p1, p2, g1, g2, counts)

    xs, wg = _dispatch_sc(x, s1, s2, g1w, g2w)
    ys = _ffn(be, xs, w1, b1.reshape(E, 1, H), w2, b2.reshape(E, 1, O), wg)
    out = _combine_sc(ys, s1, s2)
    return out, loss[0, 0]


# speedup vs baseline: 1.9533x; 1.1049x over previous
import functools

import jax
import jax.numpy as jnp
from jax import lax
from jax.experimental import pallas as pl
from jax.experimental.pallas import tpu as pltpu
from jax.experimental.pallas import tpu_sc as plsc

N = 2048
D = 768
H = 3072
O = 768
E = 8
GB = 256
NGB = N // GB
B = 512
NBLK = 16
S = NBLK * B
NW = 32
TPW = N // NW


def _gating_body(x_ref, gw9_ref, eps_ref,
                 s1_ref, s2_ref, g1w_ref, g2w_ref, be_ref, loss_ref,
                 carry, tmp, e1s, e2s):
    i = pl.program_id(0)

    @pl.when(i == 0)
    def _():
        carry[...] = jnp.zeros_like(carry)
        tmp[...] = jnp.zeros_like(tmp)

    kt = (((1,), (1,)), ((), ()))
    xb = x_ref[...].astype(jnp.bfloat16)
    gn = lax.dot_general(xb, gw9_ref[...].astype(jnp.bfloat16), kt,
                         preferred_element_type=jnp.float32)
    g = gn[:, :E]
    nz = gn[:, E:E + 1]
    sp = jnp.maximum(nz, 0.0) + jnp.log(1.0 + jnp.exp(-jnp.abs(nz)))
    h = g + eps_ref[...] * sp

    neg_inf = jnp.float32(-jnp.inf)
    m1 = jnp.min(h, axis=1, keepdims=True)
    m2 = jnp.min(jnp.where(h == m1, jnp.float32(jnp.inf), h), axis=1,
                 keepdims=True)
    h = jnp.where((h == m1) | (h == m2), neg_inf, h)

    mx = jnp.max(h, axis=1, keepdims=True)
    p = jnp.exp(h - mx)
    L = p / jnp.sum(p, axis=1, keepdims=True)
    tmp[...] += jnp.sum(L, axis=0, keepdims=True)

    lane = lax.broadcasted_iota(jnp.int32, (GB, E), 1)
    M1 = jnp.max(L, axis=1, keepdims=True)
    i1 = jnp.min(jnp.where(L == M1, lane, E), axis=1, keepdims=True)
    L2 = jnp.where(lane == i1, -1.0, L)
    M2 = jnp.max(L2, axis=1, keepdims=True)
    i2 = jnp.min(jnp.where(L2 == M2, lane, E), axis=1, keepdims=True)

    r = (jnp.equal(lane, i1) | jnp.equal(lane, i2)).astype(jnp.bfloat16)
    row = lax.broadcasted_iota(jnp.int32, (GB, GB), 0)
    col = lax.broadcasted_iota(jnp.int32, (GB, GB), 1)
    tri = (row > col).astype(jnp.bfloat16)
    posx = jnp.dot(tri, r, preferred_element_type=jnp.float32)
    pos = posx + carry[...]
    p1 = jnp.sum(jnp.where(lane == i1, pos, 0.0), axis=1, keepdims=True)
    p2 = jnp.sum(jnp.where(lane == i2, pos, 0.0), axis=1, keepdims=True)
    carry[...] += jnp.sum(r.astype(jnp.float32), axis=0, keepdims=True)

    rows_i = pl.ds(i * GB, GB)
    s1_ref[rows_i, :] = p1.astype(jnp.int32)
    s2_ref[rows_i, :] = p2.astype(jnp.int32)
    g1w_ref[rows_i, :] = jnp.broadcast_to(M1, (GB, 128))
    g2w_ref[rows_i, :] = jnp.broadcast_to(M2, (GB, 128))
    e1s[rows_i, :] = i1
    e2s[rows_i, :] = i2

    t = tmp[...]
    mu = jnp.sum(t) / E
    var = jnp.sum((t - mu) ** 2) / (E - 1)
    loss_ref[...] = jnp.full((1, 1), var / (mu * mu), jnp.float32)

    @pl.when(i == NGB - 1)
    def _():
        cnt = carry[...].astype(jnp.int32)
        nblk = (cnt + (B - 1)) >> 9
        nblk8 = jnp.broadcast_to(nblk, (E, E))
        re8 = lax.broadcasted_iota(jnp.int32, (E, E), 0)
        ce8 = lax.broadcasted_iota(jnp.int32, (E, E), 1)
        starts = jnp.sum(jnp.where(ce8 < re8, nblk8, 0), axis=1,
                         keepdims=True)
        off = starts * B

        bid = lax.broadcasted_iota(jnp.int32, (NBLK, E), 0)
        st_b = jnp.broadcast_to(starts.reshape(1, E), (NBLK, E))
        be = jnp.sum((bid >= st_b).astype(jnp.int32), axis=1,
                     keepdims=True) - 1
        used = jnp.sum(nblk)
        bcol = lax.broadcasted_iota(jnp.int32, (NBLK, 1), 0)
        be_ref[...] = (jnp.clip(be, 0, E - 1)
                       + 8 * (bcol >= used).astype(jnp.int32))

        laneN = lax.broadcasted_iota(jnp.int32, (N, E), 1)
        offrow = jnp.broadcast_to(off.reshape(1, E), (N, E))
        o1 = jnp.sum(jnp.where(laneN == e1s[...], offrow, 0), axis=1,
                     keepdims=True)
        o2 = jnp.sum(jnp.where(laneN == e2s[...], offrow, 0), axis=1,
                     keepdims=True)
        s1_ref[...] = s1_ref[...] + o1
        s2_ref[...] = s2_ref[...] + o2


def _gating(x, gw9, eps):
    cst = lambda shape: pl.BlockSpec(shape, lambda i: tuple(0 for _ in shape))
    return pl.pallas_call(
        _gating_body,
        grid=(NGB,),
        in_specs=[
            pl.BlockSpec((GB, D), lambda i: (i, 0)),
            pl.BlockSpec((E + 1, D), lambda i: (0, 0)),
            pl.BlockSpec((GB, E), lambda i: (i, 0)),
        ],
        out_specs=[cst((N, 1)), cst((N, 1)), cst((N, 128)), cst((N, 128)),
                   cst((NBLK, 1)), cst((1, 1))],
        out_shape=[
            jax.ShapeDtypeStruct((N, 1), jnp.int32),
            jax.ShapeDtypeStruct((N, 1), jnp.int32),
            jax.ShapeDtypeStruct((N, 128), jnp.float32),
            jax.ShapeDtypeStruct((N, 128), jnp.float32),
            jax.ShapeDtypeStruct((NBLK, 1), jnp.int32),
            jax.ShapeDtypeStruct((1, 1), jnp.float32),
        ],
        scratch_shapes=[pltpu.VMEM((1, E), jnp.float32),
                        pltpu.VMEM((1, E), jnp.float32),
                        pltpu.VMEM((N, 1), jnp.int32),
                        pltpu.VMEM((N, 1), jnp.int32)],
    )(x, gw9, eps)


def _dispatch_sc(x, s1, s2, g1w, g2w):
    mesh = plsc.VectorSubcoreMesh(core_axis_name="c", subcore_axis_name="s")

    @functools.partial(
        pl.kernel,
        out_type=(jax.ShapeDtypeStruct((S, D), jnp.float32),
                  jax.ShapeDtypeStruct((S, 128), jnp.float32)),
        mesh=mesh,
        scratch_types=[
            pltpu.VMEM((TPW,), jnp.int32),
            pltpu.VMEM((TPW,), jnp.int32),
            pltpu.VMEM((TPW, D), jnp.float32),
            pltpu.VMEM((TPW, 128), jnp.float32),
            pltpu.VMEM((TPW, 128), jnp.float32),
            pltpu.SemaphoreType.DMA,
        ],
    )
    def k(x_hbm, s1_hbm, s2_hbm, g1w_hbm, g2w_hbm, xs_hbm, wg_hbm,
          i1v, i2v, xv, g1v, g2v, sem):
        wid = lax.axis_index("s") * 2 + lax.axis_index("c")
        base = wid * TPW
        c1 = pltpu.async_copy(s1_hbm.at[pl.ds(base, TPW)], i1v, sem)
        c2 = pltpu.async_copy(s2_hbm.at[pl.ds(base, TPW)], i2v, sem)
        c3 = pltpu.async_copy(x_hbm.at[pl.ds(base, TPW)], xv, sem)
        c4 = pltpu.async_copy(g1w_hbm.at[pl.ds(base, TPW)], g1v, sem)
        c5 = pltpu.async_copy(g2w_hbm.at[pl.ds(base, TPW)], g2v, sem)
        for c in (c1, c2, c3, c4, c5):
            c.wait()
        d1 = pltpu.async_copy(xv, xs_hbm.at[i1v], sem)
        d2 = pltpu.async_copy(xv, xs_hbm.at[i2v], sem)
        d3 = pltpu.async_copy(g1v, wg_hbm.at[i1v], sem)
        d4 = pltpu.async_copy(g2v, wg_hbm.at[i2v], sem)
        for d in (d1, d2, d3, d4):
            d.wait()

    return k(x, s1, s2, g1w, g2w)


def _ffn_body(be_ref, xs_ref, w1_ref, b1_ref, w2_ref, b2_ref, wg_ref, ys_ref):
    b = pl.program_id(0)

    @pl.when(be_ref[b] < E)
    def _():
        kt = (((1,), (1,)), ((), ()))
        xb = xs_ref[...].astype(jnp.bfloat16)
        w1b = w1_ref[0].astype(jnp.bfloat16)
        h = lax.dot_general(xb, w1b, kt, preferred_element_type=jnp.float32)
        h = jnp.maximum(h + b1_ref[0], 0.0).astype(jnp.bfloat16)
        w2b = w2_ref[0].astype(jnp.bfloat16)
        y = lax.dot_general(h, w2b, kt, preferred_element_type=jnp.float32)
        ys_ref[...] = (y + b2_ref[0]) * wg_ref[:, :1]


def _ffn(be, xs, w1t, b1, w2t, b2, wg):
    grid_spec = pltpu.PrefetchScalarGridSpec(
        num_scalar_prefetch=1,
        grid=(NBLK,),
        in_specs=[
            pl.BlockSpec((B, D), lambda b, be: (b, 0)),
            pl.BlockSpec((1, H, D), lambda b, be: (be[b] & 7, 0, 0)),
            pl.BlockSpec((1, 1, H), lambda b, be: (be[b] & 7, 0, 0)),
            pl.BlockSpec((1, O, H), lambda b, be: (be[b] & 7, 0, 0)),
            pl.BlockSpec((1, 1, O), lambda b, be: (be[b] & 7, 0, 0)),
            pl.BlockSpec((B, 128), lambda b, be: (b, 0)),
        ],
        out_specs=pl.BlockSpec((B, O), lambda b, be: (b, 0)),
    )
    return pl.pallas_call(
        _ffn_body,
        grid_spec=grid_spec,
        out_shape=jax.ShapeDtypeStruct((S, O), jnp.float32),
    )(be, xs, w1t, b1, w2t, b2, wg)


def _combine_sc(ys, s1, s2):
    mesh = plsc.VectorSubcoreMesh(core_axis_name="c", subcore_axis_name="s")

    @functools.partial(
        pl.kernel,
        out_type=jax.ShapeDtypeStruct((N, O), jnp.float32),
        mesh=mesh,
        scratch_types=[
            pltpu.VMEM((TPW,), jnp.int32),
            pltpu.VMEM((TPW,), jnp.int32),
            pltpu.VMEM((TPW, O), jnp.float32),
            pltpu.VMEM((TPW, O), jnp.float32),
            pltpu.SemaphoreType.DMA,
        ],
    )
    def k(ys_hbm, s1_hbm, s2_hbm, out_hbm, i1v, i2v, av, bv, sem):
        wid = lax.axis_index("s") * 2 + lax.axis_index("c")
        base = wid * TPW
        c1 = pltpu.async_copy(s1_hbm.at[pl.ds(base, TPW)], i1v, sem)
        c2 = pltpu.async_copy(s2_hbm.at[pl.ds(base, TPW)], i2v, sem)
        c1.wait()
        c2.wait()
        g1 = pltpu.async_copy(ys_hbm.at[i1v], av, sem)
        g2 = pltpu.async_copy(ys_hbm.at[i2v], bv, sem)
        g1.wait()
        g2.wait()

        @pl.loop(0, TPW)
        def _(j):
            for c in range(0, O, 16):
                av.at[j, pl.ds(c, 16)][...] = (av[j, pl.ds(c, 16)]
                                               + bv[j, pl.ds(c, 16)])

        pltpu.sync_copy(av, out_hbm.at[pl.ds(base, TPW)])

    return k(ys, s1, s2)


def kernel(x, gate_w, noise_w, w1, b1, w2, b2, noise_eps):
    gw9 = jnp.concatenate([gate_w, noise_w], axis=0)
    s1_2d, s2_2d, g1w, g2w, be_2d, loss = _gating(x, gw9, noise_eps)
    s1 = s1_2d.reshape(N)
    s2 = s2_2d.reshape(N)
    be = be_2d.reshape(NBLK)

    xs, wg = _dispatch_sc(x, s1, s2, g1w, g2w)
    ys = _ffn(be, xs, w1, b1.reshape(E, 1, H), w2, b2.reshape(E, 1, O), wg)
    out = _combine_sc(ys, s1, s2)
    return out, loss[0, 0]
